# Initial kernel scaffold; baseline (speedup 1.0000x reference)
import jax
import jax.numpy as jnp
from jax.experimental import pallas as pl


def kernel(x, node_depth, edge_index, edge_attr, batch, type_emb, attr_emb, depth_emb, conv1_W, conv1_b, attn1_W, attn1_b, conv2_W, conv2_b, attn2_W, attn2_b, conv3_W, conv3_b, attn3_W, attn3_b, lin1_W, lin1_b, lin2_W, lin2_b, pred_W, pred_b):
    raise NotImplementedError("write your pallas kernel here")


# speedup vs baseline: 41.4904x; 41.4904x over previous
import functools
from math import ceil

import jax
import jax.numpy as jnp
from jax import lax
from jax.experimental import pallas as pl
from jax.experimental.pallas import tpu as pltpu
from jax.experimental.pallas import tpu_sc as plsc

_N = 10000
_E = 320000
_B = 100
_NPER = 100
_D = 128
_MAXDEPTH = 20

_NW = 32
_GCH = 64
_NPAD = 10240
_ROWS = 3 * _NPAD
_RPW = _ROWS // _NW
_GN = _RPW // _GCH

_ECH = 128
_CSLOTS = _B * _NPER * _NPER
_CPAD = 1000448
_CHALF = _CPAD // 2
_CBUF = _CHALF + 16
_CSLICE = _CHALF // 16
_EPT = 20224
_ECN = _EPT // _ECH
_EPAD = _EPT * 16


def _sc_gather_scatter(tbl, idx_all, tpad):
    mesh = plsc.VectorSubcoreMesh(core_axis_name="c", subcore_axis_name="s")

    @functools.partial(
        pl.kernel,
        mesh=mesh,
        out_type=[
            jax.ShapeDtypeStruct((_ROWS, _D), jnp.float32),
            jax.ShapeDtypeStruct((2, 16, _CSLICE), jnp.float32),
        ],
        scratch_types=[
            pltpu.VMEM((_GCH,), jnp.int32),
            pltpu.VMEM((_GCH, _D), jnp.float32),
            pltpu.VMEM((_ECH,), jnp.int32),
            pltpu.VMEM((_ECH,), jnp.float32),
            pltpu.VMEM((_CSLICE,), jnp.float32),
            pltpu.VMEM((48,), jnp.int32),
            pltpu.VMEM((48,), jnp.float32),
            pltpu.VMEM_SHARED((_CBUF,), jnp.float32),
            pltpu.SemaphoreType.DMA,
        ],
    )
    def k(tbl_hbm, idx_hbm, tpad_hbm, h3_hbm, c2_hbm,
          idx_v, rows_v, eidx_v, val_v, zv, bi, bv, cshared, gsem):
        cid = lax.axis_index("c")
        sid = lax.axis_index("s")
        wid = sid * 2 + cid
        lo = cid * _CHALF
        i32 = jnp.int32
        f32 = jnp.float32
        ones16 = jnp.ones((16,), f32)

        def zbody(i, carry):
            zv[pl.ds(i * 16, 16)] = jnp.zeros((16,), f32)
            return carry
        lax.fori_loop(0, _CSLICE // 16, zbody, 0)
        pltpu.sync_copy(zv, cshared.at[pl.ds(sid * _CSLICE, _CSLICE)])
        plsc.subcore_barrier()
        ii = lax.iota(i32, 16)
        one_i = jnp.ones((16,), i32)
        zero_i = jnp.zeros((16,), i32)
        dump_v = jnp.full((16,), _CHALF, i32)
        zero_f = jnp.zeros((16,), f32)
        gm = [jnp.where(ii >= 16 - s, one_i, zero_i) for s in range(1, 16)]

        def sbody(c, carry):
            off = sid * _EPT + c * _ECH
            pltpu.sync_copy(tpad_hbm.at[pl.ds(off, _ECH)], eidx_v)
            for j in range(_ECH // 16):
                tv = eidx_v[pl.ds(j * 16, 16)]
                bi[pl.ds(0, 16)] = tv
                bi[pl.ds(16, 16)] = tv
                cnt = one_i
                bad = zero_i
                for s in range(1, 16):
                    w = bi[pl.ds(s, 16)]
                    eqi = jnp.where(tv == w, one_i, zero_i)
                    cnt = cnt + eqi
                    bad = bad + eqi * gm[s - 1]
                inlo = jnp.where(tv >= lo, zero_i, one_i)
                inhi = jnp.where(tv < lo + _CHALF, zero_i, one_i)
                disq = bad + inlo + inhi
                eidx_v[pl.ds(j * 16, 16)] = jnp.where(disq == zero_i,
                                                      tv - lo, dump_v)
                val_v[pl.ds(j * 16, 16)] = jnp.where(bad == zero_i,
                                                     cnt.astype(f32), zero_f)
            pltpu.sync_copy(val_v, cshared.at[eidx_v], add=True)
            return carry
        lax.fori_loop(0, _ECN, sbody, 0)
        plsc.subcore_barrier()

        pltpu.sync_copy(cshared.at[pl.ds(sid * _CSLICE, _CSLICE)], zv)
        pltpu.sync_copy(zv, c2_hbm.at[cid, sid])

        def gbody(c, carry):
            off = wid * _RPW + c * _GCH
            pltpu.sync_copy(idx_hbm.at[pl.ds(off, _GCH)], idx_v)
            pltpu.async_copy(tbl_hbm.at[idx_v], rows_v, gsem).wait()
            pltpu.sync_copy(rows_v, h3_hbm.at[pl.ds(off, _GCH)])
            return carry
        lax.fori_loop(0, _GN, gbody, 0)

    return k(tbl, idx_all, tpad)


def _layer(H, C, W, cb, aW, ab, n, k):
    f32 = jnp.float32
    deg = jnp.sum(C, axis=1, keepdims=True) + 1.0
    dinv = lax.rsqrt(deg)
    rdeg = 1.0 / deg

    Hh = jnp.dot(H, W, preferred_element_type=f32, precision=lax.Precision.HIGHEST)
    agg = dinv * jnp.dot(C, dinv * Hh, preferred_element_type=f32, precision=lax.Precision.HIGHEST)
    Hc = jnp.maximum(agg + Hh * rdeg + cb, 0.0)

    sh = jnp.dot(Hc, aW, preferred_element_type=f32, precision=lax.Precision.HIGHEST)
    s2d = dinv * jnp.dot(C, dinv * sh, preferred_element_type=f32, precision=lax.Precision.HIGHEST) \
        + sh * rdeg + ab

    eyeN = (lax.broadcasted_iota(jnp.int32, (n, n), 0)
            == lax.broadcasted_iota(jnp.int32, (n, n), 1)).astype(f32)
    st = lax.dot_general(s2d, eyeN, (((0,), (0,)), ((), ())),
                         preferred_element_type=f32, precision=lax.Precision.HIGHEST)
    Sj = s2d * jnp.ones((n, n), f32)
    Si = st * jnp.ones((n, n), f32)
    ij = lax.broadcasted_iota(jnp.int32, (n, n), 0)
    ii = lax.broadcasted_iota(jnp.int32, (n, n), 1)
    T = (Sj > Si) | ((Sj == Si) & (ij < ii))
    rank = jnp.sum(T.astype(f32), axis=0, keepdims=True)

    P = (lax.broadcasted_iota(jnp.int32, (k, n), 0).astype(f32)
         == jnp.ones((k, 1), f32) * rank).astype(f32)

    sk = jnp.dot(P, s2d, preferred_element_type=f32, precision=lax.Precision.HIGHEST)
    Hn = jnp.dot(P, Hc, preferred_element_type=f32, precision=lax.Precision.HIGHEST) * jnp.tanh(sk)
    CP = lax.dot_general(C, P, (((1,), (1,)), ((), ())),
                         preferred_element_type=f32, precision=lax.Precision.HIGHEST)
    Cn = jnp.dot(P, CP, preferred_element_type=f32, precision=lax.Precision.HIGHEST)

    mx = jnp.max(Hn, axis=0, keepdims=True)
    mn = jnp.sum(Hn, axis=0, keepdims=True) * (1.0 / k)
    return Hn, Cn, mx, mn


def _gnn_body(ha, hb, hc, c2, c1W, c1b, a1W, a1b, c2W, c2b, a2W, a2b,
              c3W, c3b, a3W, a3b, out):
    H = ha[0] + hb[0] + hc[0]
    C = c2[0]
    n = _NPER
    acc = None
    Ws = ((c1W[...], c1b[...], a1W[...], a1b[...]),
          (c2W[...], c2b[...], a2W[...], a2b[...]),
          (c3W[...], c3b[...], a3W[...], a3b[...]))
    for (cW, cb, aW, ab) in Ws:
        k = int(ceil(0.5 * n))
        H, C, mx, mn = _layer(H, C, cW, cb, aW, ab, n, k)
        r = jnp.concatenate([mx, mn], axis=1)
        acc = r if acc is None else acc + r
        n = k
    out[...] = acc[None]


def _mlp_body(g, w1, b1, w2, b2, out):
    f32 = jnp.float32
    h = jnp.maximum(jnp.dot(g[...], w1[...], preferred_element_type=f32, precision=lax.Precision.HIGHEST)
                    + b1[...], 0.0)
    out[...] = jnp.maximum(jnp.dot(h, w2[...], preferred_element_type=f32, precision=lax.Precision.HIGHEST)
                           + b2[...], 0.0)


def _pred_body(gg, pW, pb, out):
    out[...] = (jnp.dot(gg[...], pW[0], preferred_element_type=jnp.float32, precision=lax.Precision.HIGHEST)
                + pb[0])[None]


def kernel(x, node_depth, edge_index, edge_attr, batch, type_emb, attr_emb,
           depth_emb, conv1_W, conv1_b, attn1_W, attn1_b, conv2_W, conv2_b,
           attn2_W, attn2_b, conv3_W, conv3_b, attn3_W, attn3_b, lin1_W,
           lin1_b, lin2_W, lin2_b, pred_W, pred_b):
    f32 = jnp.float32
    i32 = jnp.int32

    src = edge_index[0].astype(i32)
    dst = edge_index[1].astype(i32)
    g = src // _NPER
    t = g * (_NPER * _NPER) + (dst % _NPER) * _NPER + (src % _NPER)
    tpad = jnp.full((_EPAD,), _CSLOTS, i32).at[: _E].set(t)

    nt = type_emb.shape[0]
    na = attr_emb.shape[0]
    depth = jnp.clip(node_depth[:, 0], 0, _MAXDEPTH).astype(i32)
    i0 = jnp.zeros((_NPAD,), i32).at[: _N].set(x[:, 0].astype(i32))
    i1 = jnp.zeros((_NPAD,), i32).at[: _N].set(nt + x[:, 1].astype(i32))
    i2 = jnp.zeros((_NPAD,), i32).at[: _N].set(nt + na + depth)
    idx_all = jnp.concatenate([i0, i1, i2])
    tbl = jnp.concatenate([type_emb, attr_emb, depth_emb], axis=0)

    h3, c2 = _sc_gather_scatter(tbl, idx_all, tpad)

    ha = h3[0:_N].reshape(_B, _NPER, _D)
    hb = h3[_NPAD:_NPAD + _N].reshape(_B, _NPER, _D)
    hc = h3[2 * _NPAD:2 * _NPAD + _N].reshape(_B, _NPER, _D)
    c_lin = c2.reshape(_CPAD)[:_CSLOTS].reshape(_B, _NPER, _NPER)

    biases = [b.reshape(1, -1) for b in
              (conv1_b, attn1_b, conv2_b, attn2_b, conv3_b, attn3_b,
               lin1_b, lin2_b)]
    (c1b, a1b, c2b, a2b, c3b, a3b, l1b, l2b) = biases

    full = lambda arr: pl.BlockSpec(arr.shape, lambda gi: (0,) * arr.ndim)

    gsum = pl.pallas_call(
        _gnn_body,
        grid=(_B,),
        in_specs=[
            pl.BlockSpec((1, _NPER, _D), lambda gi: (gi, 0, 0)),
            pl.BlockSpec((1, _NPER, _D), lambda gi: (gi, 0, 0)),
            pl.BlockSpec((1, _NPER, _D), lambda gi: (gi, 0, 0)),
            pl.BlockSpec((1, _NPER, _NPER), lambda gi: (gi, 0, 0)),
            full(conv1_W), full(c1b), full(attn1_W), full(a1b),
            full(conv2_W), full(c2b), full(attn2_W), full(a2b),
            full(conv3_W), full(c3b), full(attn3_W), full(a3b),
        ],
        out_specs=pl.BlockSpec((1, 1, 2 * _D), lambda gi: (gi, 0, 0)),
        out_shape=jax.ShapeDtypeStruct((_B, 1, 2 * _D), f32),
    )(ha, hb, hc, c_lin, conv1_W, c1b, attn1_W, a1b,
      conv2_W, c2b, attn2_W, a2b, conv3_W, c3b, attn3_W, a3b)
    gsum = gsum.reshape(_B, 2 * _D)

    gg = pl.pallas_call(
        _mlp_body,
        out_shape=jax.ShapeDtypeStruct((_B, _D), f32),
    )(gsum, lin1_W, l1b, lin2_W, l2b)

    SEQ, _, VOCAB = pred_W.shape
    preds = pl.pallas_call(
        _pred_body,
        grid=(SEQ,),
        in_specs=[
            pl.BlockSpec((_B, _D), lambda s: (0, 0)),
            pl.BlockSpec((1, _D, VOCAB), lambda s: (s, 0, 0)),
            pl.BlockSpec((1, 1, VOCAB), lambda s: (s, 0, 0)),
        ],
        out_specs=pl.BlockSpec((1, _B, VOCAB), lambda s: (s, 0, 0)),
        out_shape=jax.ShapeDtypeStruct((SEQ, _B, VOCAB), f32),
    )(gg, pred_W, pred_b.reshape(SEQ, 1, VOCAB))

    return preds


# speedup vs baseline: 44.1048x; 1.0630x over previous
import functools
from math import ceil

import jax
import jax.numpy as jnp
from jax import lax
from jax.experimental import pallas as pl
from jax.experimental.pallas import tpu as pltpu
from jax.experimental.pallas import tpu_sc as plsc

_N = 10000
_E = 320000
_B = 100
_NPER = 100
_D = 128
_MAXDEPTH = 20

_NW = 32
_GCH = 64
_NPAD = 10240
_ROWS = 3 * _NPAD
_RPW = _ROWS // _NW
_GN = _RPW // _GCH

_ECH = 128
_CSLOTS = _B * _NPER * _NPER
_CPAD = 1000448
_CHALF = _CPAD // 2
_CBUF = _CHALF + 16
_CSLICE = _CHALF // 16
_EPT = 20224
_ECN = _EPT // _ECH
_EPAD = _EPT * 16


def _sc_gather_scatter(tbl, idx_all, tpad):
    mesh = plsc.VectorSubcoreMesh(core_axis_name="c", subcore_axis_name="s")

    @functools.partial(
        pl.kernel,
        mesh=mesh,
        out_type=[
            jax.ShapeDtypeStruct((_ROWS, _D), jnp.float32),
            jax.ShapeDtypeStruct((2, 16, _CSLICE), jnp.float32),
        ],
        scratch_types=[
            pltpu.VMEM((_GCH,), jnp.int32),
            pltpu.VMEM((_GCH, _D), jnp.float32),
            pltpu.VMEM((_ECH,), jnp.int32),
            pltpu.VMEM((_ECH,), jnp.float32),
            pltpu.VMEM((_CSLICE,), jnp.float32),
            pltpu.VMEM((48,), jnp.int32),
            pltpu.VMEM((48,), jnp.float32),
            pltpu.VMEM_SHARED((_CBUF,), jnp.float32),
            pltpu.SemaphoreType.DMA,
        ],
    )
    def k(tbl_hbm, idx_hbm, tpad_hbm, h3_hbm, c2_hbm,
          idx_v, rows_v, eidx_v, val_v, zv, bi, bv, cshared, gsem):
        cid = lax.axis_index("c")
        sid = lax.axis_index("s")
        wid = sid * 2 + cid
        lo = cid * _CHALF
        i32 = jnp.int32
        f32 = jnp.float32
        ones16 = jnp.ones((16,), f32)

        def zbody(i, carry):
            zv[pl.ds(i * 16, 16)] = jnp.zeros((16,), f32)
            return carry
        lax.fori_loop(0, _CSLICE // 16, zbody, 0)
        pltpu.sync_copy(zv, cshared.at[pl.ds(sid * _CSLICE, _CSLICE)])
        plsc.subcore_barrier()
        ii = lax.iota(i32, 16)
        one_i = jnp.ones((16,), i32)
        zero_i = jnp.zeros((16,), i32)
        dump_v = jnp.full((16,), _CHALF, i32)
        zero_f = jnp.zeros((16,), f32)
        gm = [jnp.where(ii >= 16 - s, one_i, zero_i) for s in range(1, 16)]

        def sbody(c, carry):
            off = sid * _EPT + c * _ECH
            pltpu.sync_copy(tpad_hbm.at[pl.ds(off, _ECH)], eidx_v)
            for j in range(_ECH // 16):
                tv = eidx_v[pl.ds(j * 16, 16)]
                bi[pl.ds(0, 16)] = tv
                bi[pl.ds(16, 16)] = tv
                cnt = one_i
                bad = zero_i
                for s in range(1, 16):
                    w = bi[pl.ds(s, 16)]
                    eqi = jnp.where(tv == w, one_i, zero_i)
                    cnt = cnt + eqi
                    bad = bad + eqi * gm[s - 1]
                inlo = jnp.where(tv >= lo, zero_i, one_i)
                inhi = jnp.where(tv < lo + _CHALF, zero_i, one_i)
                disq = bad + inlo + inhi
                eidx_v[pl.ds(j * 16, 16)] = jnp.where(disq == zero_i,
                                                      tv - lo, dump_v)
                val_v[pl.ds(j * 16, 16)] = jnp.where(bad == zero_i,
                                                     cnt.astype(f32), zero_f)
            pltpu.sync_copy(val_v, cshared.at[eidx_v], add=True)
            return carry
        lax.fori_loop(0, _ECN, sbody, 0)
        plsc.subcore_barrier()

        pltpu.sync_copy(cshared.at[pl.ds(sid * _CSLICE, _CSLICE)], zv)
        pltpu.sync_copy(zv, c2_hbm.at[cid, sid])

        def gbody(c, carry):
            off = wid * _RPW + c * _GCH
            pltpu.sync_copy(idx_hbm.at[pl.ds(off, _GCH)], idx_v)
            pltpu.async_copy(tbl_hbm.at[idx_v], rows_v, gsem).wait()
            pltpu.sync_copy(rows_v, h3_hbm.at[pl.ds(off, _GCH)])
            return carry
        lax.fori_loop(0, _GN, gbody, 0)

    return k(tbl, idx_all, tpad)


def _layer(H, C, W, cb, aW, ab, n, k):
    f32 = jnp.float32
    deg = jnp.sum(C, axis=1, keepdims=True) + 1.0
    dinv = 1.0 / jnp.sqrt(deg)
    rdeg = 1.0 / deg

    Hh = jnp.dot(H, W, preferred_element_type=f32)
    agg = dinv * jnp.dot(C, dinv * Hh, preferred_element_type=f32, precision=lax.Precision.HIGHEST)
    Hc = jnp.maximum(agg + Hh * rdeg + cb, 0.0)

    sh = jnp.dot(Hc, aW, preferred_element_type=f32)
    s2d = dinv * jnp.dot(C, dinv * sh, preferred_element_type=f32, precision=lax.Precision.HIGHEST) \
        + sh * rdeg + ab

    eyeN = (lax.broadcasted_iota(jnp.int32, (n, n), 0)
            == lax.broadcasted_iota(jnp.int32, (n, n), 1)).astype(f32)
    st = lax.dot_general(s2d, eyeN, (((0,), (0,)), ((), ())),
                         preferred_element_type=f32, precision=lax.Precision.HIGHEST)
    Sj = s2d * jnp.ones((n, n), f32)
    Si = st * jnp.ones((n, n), f32)
    ij = lax.broadcasted_iota(jnp.int32, (n, n), 0)
    ii = lax.broadcasted_iota(jnp.int32, (n, n), 1)
    T = (Sj > Si) | ((Sj == Si) & (ij < ii))
    rank = jnp.sum(T.astype(f32), axis=0, keepdims=True)

    P = (lax.broadcasted_iota(jnp.int32, (k, n), 0).astype(f32)
         == jnp.ones((k, 1), f32) * rank).astype(f32)

    sk = jnp.dot(P, s2d, preferred_element_type=f32, precision=lax.Precision.HIGHEST)
    Hn = jnp.dot(P, Hc, preferred_element_type=f32, precision=lax.Precision.HIGHEST) * jnp.tanh(sk)
    CP = lax.dot_general(C, P, (((1,), (1,)), ((), ())),
                         preferred_element_type=f32, precision=lax.Precision.HIGHEST)
    Cn = jnp.dot(P, CP, preferred_element_type=f32, precision=lax.Precision.HIGHEST)

    mx = jnp.max(Hn, axis=0, keepdims=True)
    mn = jnp.sum(Hn, axis=0, keepdims=True) / f32(k)
    return Hn, Cn, mx, mn


def _gnn_body(ha, hb, hc, c2, c1W, c1b, a1W, a1b, c2W, c2b, a2W, a2b,
              c3W, c3b, a3W, a3b, out):
    H = ha[0] + hb[0] + hc[0]
    C = c2[0]
    n = _NPER
    acc = None
    Ws = ((c1W[...], c1b[...], a1W[...], a1b[...]),
          (c2W[...], c2b[...], a2W[...], a2b[...]),
          (c3W[...], c3b[...], a3W[...], a3b[...]))
    for (cW, cb, aW, ab) in Ws:
        k = int(ceil(0.5 * n))
        H, C, mx, mn = _layer(H, C, cW, cb, aW, ab, n, k)
        r = jnp.concatenate([mx, mn], axis=1)
        acc = r if acc is None else acc + r
        n = k
    out[...] = acc[None]


def _mlp_body(g, w1, b1, w2, b2, out):
    f32 = jnp.float32
    h = jnp.maximum(jnp.dot(g[...], w1[...], preferred_element_type=f32)
                    + b1[...], 0.0)
    out[...] = jnp.maximum(jnp.dot(h, w2[...], preferred_element_type=f32)
                           + b2[...], 0.0)


def _pred_body(gg, pW, pb, out):
    out[...] = (jnp.dot(gg[...], pW[0], preferred_element_type=jnp.float32)
                + pb[0])[None]


def kernel(x, node_depth, edge_index, edge_attr, batch, type_emb, attr_emb,
           depth_emb, conv1_W, conv1_b, attn1_W, attn1_b, conv2_W, conv2_b,
           attn2_W, attn2_b, conv3_W, conv3_b, attn3_W, attn3_b, lin1_W,
           lin1_b, lin2_W, lin2_b, pred_W, pred_b):
    f32 = jnp.float32
    i32 = jnp.int32

    src = edge_index[0].astype(i32)
    dst = edge_index[1].astype(i32)
    g = src // _NPER
    t = g * (_NPER * _NPER) + (dst % _NPER) * _NPER + (src % _NPER)
    tpad = jnp.full((_EPAD,), _CSLOTS, i32).at[: _E].set(t)

    nt = type_emb.shape[0]
    na = attr_emb.shape[0]
    depth = jnp.clip(node_depth[:, 0], 0, _MAXDEPTH).astype(i32)
    i0 = jnp.zeros((_NPAD,), i32).at[: _N].set(x[:, 0].astype(i32))
    i1 = jnp.zeros((_NPAD,), i32).at[: _N].set(nt + x[:, 1].astype(i32))
    i2 = jnp.zeros((_NPAD,), i32).at[: _N].set(nt + na + depth)
    idx_all = jnp.concatenate([i0, i1, i2])
    tbl = jnp.concatenate([type_emb, attr_emb, depth_emb], axis=0)

    h3, c2 = _sc_gather_scatter(tbl, idx_all, tpad)

    ha = h3[0:_N].reshape(_B, _NPER, _D)
    hb = h3[_NPAD:_NPAD + _N].reshape(_B, _NPER, _D)
    hc = h3[2 * _NPAD:2 * _NPAD + _N].reshape(_B, _NPER, _D)
    c_lin = c2.reshape(_CPAD)[:_CSLOTS].reshape(_B, _NPER, _NPER)

    biases = [b.reshape(1, -1) for b in
              (conv1_b, attn1_b, conv2_b, attn2_b, conv3_b, attn3_b,
               lin1_b, lin2_b)]
    (c1b, a1b, c2b, a2b, c3b, a3b, l1b, l2b) = biases

    full = lambda arr: pl.BlockSpec(arr.shape, lambda gi: (0,) * arr.ndim)

    gsum = pl.pallas_call(
        _gnn_body,
        grid=(_B,),
        in_specs=[
            pl.BlockSpec((1, _NPER, _D), lambda gi: (gi, 0, 0)),
            pl.BlockSpec((1, _NPER, _D), lambda gi: (gi, 0, 0)),
            pl.BlockSpec((1, _NPER, _D), lambda gi: (gi, 0, 0)),
            pl.BlockSpec((1, _NPER, _NPER), lambda gi: (gi, 0, 0)),
            full(conv1_W), full(c1b), full(attn1_W), full(a1b),
            full(conv2_W), full(c2b), full(attn2_W), full(a2b),
            full(conv3_W), full(c3b), full(attn3_W), full(a3b),
        ],
        out_specs=pl.BlockSpec((1, 1, 2 * _D), lambda gi: (gi, 0, 0)),
        out_shape=jax.ShapeDtypeStruct((_B, 1, 2 * _D), f32),
    )(ha, hb, hc, c_lin, conv1_W, c1b, attn1_W, a1b,
      conv2_W, c2b, attn2_W, a2b, conv3_W, c3b, attn3_W, a3b)
    gsum = gsum.reshape(_B, 2 * _D)

    gg = pl.pallas_call(
        _mlp_body,
        out_shape=jax.ShapeDtypeStruct((_B, _D), f32),
    )(gsum, lin1_W, l1b, lin2_W, l2b)

    SEQ, _, VOCAB = pred_W.shape
    preds = pl.pallas_call(
        _pred_body,
        grid=(SEQ,),
        in_specs=[
            pl.BlockSpec((_B, _D), lambda s: (0, 0)),
            pl.BlockSpec((1, _D, VOCAB), lambda s: (s, 0, 0)),
            pl.BlockSpec((1, 1, VOCAB), lambda s: (s, 0, 0)),
        ],
        out_specs=pl.BlockSpec((1, _B, VOCAB), lambda s: (s, 0, 0)),
        out_shape=jax.ShapeDtypeStruct((SEQ, _B, VOCAB), f32),
    )(gg, pred_W, pred_b.reshape(SEQ, 1, VOCAB))

    return preds


# speedup vs baseline: 44.8227x; 1.0163x over previous
import functools
from math import ceil

import jax
import jax.numpy as jnp
from jax import lax
from jax.experimental import pallas as pl
from jax.experimental.pallas import tpu as pltpu
from jax.experimental.pallas import tpu_sc as plsc

_N = 10000
_E = 320000
_B = 100
_NPER = 100
_D = 128
_MAXDEPTH = 20

_NW = 32
_GCH = 64
_NPAD = 10240
_ROWS = 3 * _NPAD
_RPW = _ROWS // _NW
_GN = _RPW // _GCH

_ECH = 128
_CSLOTS = _B * _NPER * _NPER
_CPAD = 1000448
_CHALF = _CPAD // 2
_CBUF = _CHALF + 16
_CSLICE = _CHALF // 16
_EPT = 20224
_ECN = _EPT // _ECH
_EPAD = _EPT * 16


def _sc_gather_scatter(tbl, idx_all, tpad):
    mesh = plsc.VectorSubcoreMesh(core_axis_name="c", subcore_axis_name="s")

    @functools.partial(
        pl.kernel,
        mesh=mesh,
        out_type=[
            jax.ShapeDtypeStruct((_ROWS, _D), jnp.float32),
            jax.ShapeDtypeStruct((2, 16, _CSLICE), jnp.float32),
        ],
        scratch_types=[
            pltpu.VMEM((_GCH,), jnp.int32),
            pltpu.VMEM((_GCH, _D), jnp.float32),
            pltpu.VMEM((_ECH,), jnp.int32),
            pltpu.VMEM((_ECH,), jnp.float32),
            pltpu.VMEM((_CSLICE,), jnp.float32),
            pltpu.VMEM((48,), jnp.int32),
            pltpu.VMEM((48,), jnp.float32),
            pltpu.VMEM_SHARED((_CBUF,), jnp.float32),
            pltpu.SemaphoreType.DMA,
        ],
    )
    def k(tbl_hbm, idx_hbm, tpad_hbm, h3_hbm, c2_hbm,
          idx_v, rows_v, eidx_v, val_v, zv, bi, bv, cshared, gsem):
        cid = lax.axis_index("c")
        sid = lax.axis_index("s")
        wid = sid * 2 + cid
        lo = cid * _CHALF
        i32 = jnp.int32
        f32 = jnp.float32
        ones16 = jnp.ones((16,), f32)

        def zbody(i, carry):
            zv[pl.ds(i * 16, 16)] = jnp.zeros((16,), f32)
            return carry
        lax.fori_loop(0, _CSLICE // 16, zbody, 0)
        pltpu.sync_copy(zv, cshared.at[pl.ds(sid * _CSLICE, _CSLICE)])
        plsc.subcore_barrier()
        ii = lax.iota(i32, 16)
        one_i = jnp.ones((16,), i32)
        zero_i = jnp.zeros((16,), i32)
        dump_v = jnp.full((16,), _CHALF, i32)
        zero_f = jnp.zeros((16,), f32)
        gm = [jnp.where(ii >= 16 - s, one_i, zero_i) for s in range(1, 16)]

        def sbody(c, carry):
            off = sid * _EPT + c * _ECH
            pltpu.sync_copy(tpad_hbm.at[pl.ds(off, _ECH)], eidx_v)
            for j in range(_ECH // 16):
                tv = eidx_v[pl.ds(j * 16, 16)]
                bi[pl.ds(0, 16)] = tv
                bi[pl.ds(16, 16)] = tv
                cnt = one_i
                bad = zero_i
                for s in range(1, 16):
                    w = bi[pl.ds(s, 16)]
                    eqi = jnp.where(tv == w, one_i, zero_i)
                    cnt = cnt + eqi
                    bad = bad + eqi * gm[s - 1]
                inlo = jnp.where(tv >= lo, zero_i, one_i)
                inhi = jnp.where(tv < lo + _CHALF, zero_i, one_i)
                disq = bad + inlo + inhi
                eidx_v[pl.ds(j * 16, 16)] = jnp.where(disq == zero_i,
                                                      tv - lo, dump_v)
                val_v[pl.ds(j * 16, 16)] = jnp.where(bad == zero_i,
                                                     cnt.astype(f32), zero_f)
            pltpu.sync_copy(val_v, cshared.at[eidx_v], add=True)
            return carry
        lax.fori_loop(0, _ECN, sbody, 0)
        plsc.subcore_barrier()

        pltpu.sync_copy(cshared.at[pl.ds(sid * _CSLICE, _CSLICE)], zv)
        pltpu.sync_copy(zv, c2_hbm.at[cid, sid])

        def gbody(c, carry):
            off = wid * _RPW + c * _GCH
            pltpu.sync_copy(idx_hbm.at[pl.ds(off, _GCH)], idx_v)
            pltpu.async_copy(tbl_hbm.at[idx_v], rows_v, gsem).wait()
            pltpu.sync_copy(rows_v, h3_hbm.at[pl.ds(off, _GCH)])
            return carry
        lax.fori_loop(0, _GN, gbody, 0)

    return k(tbl, idx_all, tpad)


def _layer(H, C, W, cb, aW, ab, n, k):
    f32 = jnp.float32
    deg = jnp.sum(C, axis=1, keepdims=True) + 1.0
    dinv = 1.0 / jnp.sqrt(deg)
    rdeg = 1.0 / deg

    Hh = jnp.dot(H, W, preferred_element_type=f32)
    agg = dinv * jnp.dot(C, dinv * Hh, preferred_element_type=f32, precision=lax.Precision.HIGHEST)
    Hc = jnp.maximum(agg + Hh * rdeg + cb, 0.0)

    sh = jnp.dot(Hc, aW, preferred_element_type=f32)
    s2d = dinv * jnp.dot(C, dinv * sh, preferred_element_type=f32, precision=lax.Precision.HIGHEST) \
        + sh * rdeg + ab

    eyeN = (lax.broadcasted_iota(jnp.int32, (n, n), 0)
            == lax.broadcasted_iota(jnp.int32, (n, n), 1)).astype(f32)
    st = lax.dot_general(s2d, eyeN, (((0,), (0,)), ((), ())),
                         preferred_element_type=f32, precision=lax.Precision.HIGHEST)
    Sj = s2d * jnp.ones((n, n), f32)
    Si = st * jnp.ones((n, n), f32)
    ij = lax.broadcasted_iota(jnp.int32, (n, n), 0)
    ii = lax.broadcasted_iota(jnp.int32, (n, n), 1)
    T = (Sj > Si) | ((Sj == Si) & (ij < ii))
    rank = jnp.sum(T.astype(f32), axis=0, keepdims=True)

    P = (lax.broadcasted_iota(jnp.int32, (k, n), 0).astype(f32)
         == jnp.ones((k, 1), f32) * rank).astype(f32)

    sk = jnp.dot(P, s2d, preferred_element_type=f32, precision=lax.Precision.HIGHEST)
    Hn = jnp.dot(P, Hc, preferred_element_type=f32, precision=lax.Precision.HIGHEST) * jnp.tanh(sk)
    CP = lax.dot_general(C, P, (((1,), (1,)), ((), ())),
                         preferred_element_type=f32, precision=lax.Precision.HIGHEST)
    Cn = jnp.dot(P, CP, preferred_element_type=f32, precision=lax.Precision.HIGHEST)

    mx = jnp.max(Hn, axis=0, keepdims=True)
    mn = jnp.sum(Hn, axis=0, keepdims=True) / f32(k)
    return Hn, Cn, mx, mn


_GB = 4


def _gnn_body(ha, hb, hc, c2, c1W, c1b, a1W, a1b, c2W, c2b, a2W, a2b,
              c3W, c3b, a3W, a3b, out):
    Ws = ((c1W[...], c1b[...], a1W[...], a1b[...]),
          (c2W[...], c2b[...], a2W[...], a2b[...]),
          (c3W[...], c3b[...], a3W[...], a3b[...]))
    for g in range(_GB):
        H = ha[g] + hb[g] + hc[g]
        C = c2[g]
        n = _NPER
        acc = None
        for (cW, cb, aW, ab) in Ws:
            k = int(ceil(0.5 * n))
            H, C, mx, mn = _layer(H, C, cW, cb, aW, ab, n, k)
            r = jnp.concatenate([mx, mn], axis=1)
            acc = r if acc is None else acc + r
            n = k
        out[g] = acc


def _mlp_body(g, w1, b1, w2, b2, out):
    f32 = jnp.float32
    h = jnp.maximum(jnp.dot(g[...], w1[...], preferred_element_type=f32)
                    + b1[...], 0.0)
    out[...] = jnp.maximum(jnp.dot(h, w2[...], preferred_element_type=f32)
                           + b2[...], 0.0)


def _pred_body(gg, pW, pb, out):
    out[...] = (jnp.dot(gg[...], pW[0], preferred_element_type=jnp.float32)
                + pb[0])[None]


def kernel(x, node_depth, edge_index, edge_attr, batch, type_emb, attr_emb,
           depth_emb, conv1_W, conv1_b, attn1_W, attn1_b, conv2_W, conv2_b,
           attn2_W, attn2_b, conv3_W, conv3_b, attn3_W, attn3_b, lin1_W,
           lin1_b, lin2_W, lin2_b, pred_W, pred_b):
    f32 = jnp.float32
    i32 = jnp.int32

    src = edge_index[0].astype(i32)
    dst = edge_index[1].astype(i32)
    g = src // _NPER
    t = g * (_NPER * _NPER) + (dst % _NPER) * _NPER + (src % _NPER)
    tpad = jnp.full((_EPAD,), _CSLOTS, i32).at[: _E].set(t)

    nt = type_emb.shape[0]
    na = attr_emb.shape[0]
    depth = jnp.clip(node_depth[:, 0], 0, _MAXDEPTH).astype(i32)
    i0 = jnp.zeros((_NPAD,), i32).at[: _N].set(x[:, 0].astype(i32))
    i1 = jnp.zeros((_NPAD,), i32).at[: _N].set(nt + x[:, 1].astype(i32))
    i2 = jnp.zeros((_NPAD,), i32).at[: _N].set(nt + na + depth)
    idx_all = jnp.concatenate([i0, i1, i2])
    tbl = jnp.concatenate([type_emb, attr_emb, depth_emb], axis=0)

    h3, c2 = _sc_gather_scatter(tbl, idx_all, tpad)

    ha = h3[0:_N].reshape(_B, _NPER, _D)
    hb = h3[_NPAD:_NPAD + _N].reshape(_B, _NPER, _D)
    hc = h3[2 * _NPAD:2 * _NPAD + _N].reshape(_B, _NPER, _D)
    c_lin = c2.reshape(_CPAD)[:_CSLOTS].reshape(_B, _NPER, _NPER)

    biases = [b.reshape(1, -1) for b in
              (conv1_b, attn1_b, conv2_b, attn2_b, conv3_b, attn3_b,
               lin1_b, lin2_b)]
    (c1b, a1b, c2b, a2b, c3b, a3b, l1b, l2b) = biases

    full = lambda arr: pl.BlockSpec(arr.shape, lambda gi: (0,) * arr.ndim)

    gsum = pl.pallas_call(
        _gnn_body,
        grid=(_B // _GB,),
        in_specs=[
            pl.BlockSpec((_GB, _NPER, _D), lambda gi: (gi, 0, 0)),
            pl.BlockSpec((_GB, _NPER, _D), lambda gi: (gi, 0, 0)),
            pl.BlockSpec((_GB, _NPER, _D), lambda gi: (gi, 0, 0)),
            pl.BlockSpec((_GB, _NPER, _NPER), lambda gi: (gi, 0, 0)),
            full(conv1_W), full(c1b), full(attn1_W), full(a1b),
            full(conv2_W), full(c2b), full(attn2_W), full(a2b),
            full(conv3_W), full(c3b), full(attn3_W), full(a3b),
        ],
        out_specs=pl.BlockSpec((_GB, 1, 2 * _D), lambda gi: (gi, 0, 0)),
        out_shape=jax.ShapeDtypeStruct((_B, 1, 2 * _D), f32),
    )(ha, hb, hc, c_lin, conv1_W, c1b, attn1_W, a1b,
      conv2_W, c2b, attn2_W, a2b, conv3_W, c3b, attn3_W, a3b)
    gsum = gsum.reshape(_B, 2 * _D)

    gg = pl.pallas_call(
        _mlp_body,
        out_shape=jax.ShapeDtypeStruct((_B, _D), f32),
    )(gsum, lin1_W, l1b, lin2_W, l2b)

    SEQ, _, VOCAB = pred_W.shape
    preds = pl.pallas_call(
        _pred_body,
        grid=(SEQ,),
        in_specs=[
            pl.BlockSpec((_B, _D), lambda s: (0, 0)),
            pl.BlockSpec((1, _D, VOCAB), lambda s: (s, 0, 0)),
            pl.BlockSpec((1, 1, VOCAB), lambda s: (s, 0, 0)),
        ],
        out_specs=pl.BlockSpec((1, _B, VOCAB), lambda s: (s, 0, 0)),
        out_shape=jax.ShapeDtypeStruct((SEQ, _B, VOCAB), f32),
    )(gg, pred_W, pred_b.reshape(SEQ, 1, VOCAB))

    return preds


# speedup vs baseline: 45.1599x; 1.0075x over previous
import functools
from math import ceil

import jax
import jax.numpy as jnp
from jax import lax
from jax.experimental import pallas as pl
from jax.experimental.pallas import tpu as pltpu
from jax.experimental.pallas import tpu_sc as plsc

_N = 10000
_E = 320000
_B = 100
_NPER = 100
_D = 128
_MAXDEPTH = 20

_NW = 32
_GCH = 64
_NPAD = 10240
_ROWS = 3 * _NPAD
_RPW = _ROWS // _NW
_GN = _RPW // _GCH

_ECH = 128
_CSLOTS = _B * _NPER * _NPER
_CPAD = 1000448
_CHALF = _CPAD // 2
_CBUF = _CHALF + 16
_CSLICE = _CHALF // 16
_EPT = 20224
_ECN = _EPT // _ECH
_EPAD = _EPT * 16


def _sc_gather_scatter(tbl, idx_all, tpad):
    mesh = plsc.VectorSubcoreMesh(core_axis_name="c", subcore_axis_name="s")

    @functools.partial(
        pl.kernel,
        mesh=mesh,
        out_type=[
            jax.ShapeDtypeStruct((_ROWS, _D), jnp.float32),
            jax.ShapeDtypeStruct((2, 16, _CSLICE), jnp.float32),
        ],
        scratch_types=[
            pltpu.VMEM((_GCH,), jnp.int32),
            pltpu.VMEM((_GCH, _D), jnp.float32),
            pltpu.VMEM((_ECH,), jnp.int32),
            pltpu.VMEM((_ECH,), jnp.int32),
            pltpu.VMEM((_ECH,), jnp.float32),
            pltpu.VMEM((_CSLICE,), jnp.float32),
            pltpu.VMEM((48,), jnp.int32),
            pltpu.VMEM((48,), jnp.float32),
            pltpu.VMEM_SHARED((_CBUF,), jnp.float32),
            pltpu.SemaphoreType.DMA,
            pltpu.SemaphoreType.DMA,
            pltpu.SemaphoreType.DMA,
        ],
    )
    def k(tbl_hbm, idx_hbm, tpad_hbm, h3_hbm, c2_hbm,
          idx_v, rows_v, ea, eb, val_v, zv, bi, bv, cshared, gsem,
          esema, esemb):
        cid = lax.axis_index("c")
        sid = lax.axis_index("s")
        wid = sid * 2 + cid
        lo = cid * _CHALF
        i32 = jnp.int32
        f32 = jnp.float32
        ones16 = jnp.ones((16,), f32)

        def zbody(i, carry):
            zv[pl.ds(i * 16, 16)] = jnp.zeros((16,), f32)
            return carry
        lax.fori_loop(0, _CSLICE // 16, zbody, 0)
        pltpu.sync_copy(zv, cshared.at[pl.ds(sid * _CSLICE, _CSLICE)])
        plsc.subcore_barrier()
        ii = lax.iota(i32, 16)
        one_i = jnp.ones((16,), i32)
        zero_i = jnp.zeros((16,), i32)
        dump_v = jnp.full((16,), _CHALF, i32)
        zero_f = jnp.zeros((16,), f32)
        gm = [jnp.where(ii >= 16 - s, one_i, zero_i) for s in range(1, 16)]

        def dedup_scatter(ebuf):
            for j in range(_ECH // 16):
                tv = ebuf[pl.ds(j * 16, 16)]
                bi[pl.ds(0, 16)] = tv
                bi[pl.ds(16, 16)] = tv
                cnt = one_i
                bad = zero_i
                for s in range(1, 16):
                    w = bi[pl.ds(s, 16)]
                    eqi = jnp.where(tv == w, one_i, zero_i)
                    cnt = cnt + eqi
                    bad = bad + eqi * gm[s - 1]
                inlo = jnp.where(tv >= lo, zero_i, one_i)
                inhi = jnp.where(tv < lo + _CHALF, zero_i, one_i)
                disq = bad + inlo + inhi
                ebuf[pl.ds(j * 16, 16)] = jnp.where(disq == zero_i,
                                                    tv - lo, dump_v)
                val_v[pl.ds(j * 16, 16)] = jnp.where(bad == zero_i,
                                                     cnt.astype(f32), zero_f)
            pltpu.sync_copy(val_v, cshared.at[ebuf], add=True)

        base = sid * _EPT
        pltpu.async_copy(tpad_hbm.at[pl.ds(base, _ECH)], ea, esema)
        pltpu.async_copy(tpad_hbm.at[pl.ds(base + _ECH, _ECH)], eb, esemb)

        def sbody(p, carry):
            off = base + 2 * p * _ECH
            pltpu.make_async_copy(tpad_hbm.at[pl.ds(off, _ECH)],
                                  ea, esema).wait()
            dedup_scatter(ea)
            pltpu.async_copy(tpad_hbm.at[pl.ds(off + 2 * _ECH, _ECH)],
                             ea, esema)
            pltpu.make_async_copy(tpad_hbm.at[pl.ds(off + _ECH, _ECH)],
                                  eb, esemb).wait()
            dedup_scatter(eb)
            pltpu.async_copy(tpad_hbm.at[pl.ds(off + 3 * _ECH, _ECH)],
                             eb, esemb)
            return carry
        lax.fori_loop(0, _ECN // 2, sbody, 0)
        pltpu.make_async_copy(tpad_hbm.at[pl.ds(base, _ECH)], ea,
                              esema).wait()
        pltpu.make_async_copy(tpad_hbm.at[pl.ds(base, _ECH)], eb,
                              esemb).wait()
        plsc.subcore_barrier()

        pltpu.sync_copy(cshared.at[pl.ds(sid * _CSLICE, _CSLICE)], zv)
        pltpu.sync_copy(zv, c2_hbm.at[cid, sid])

        def gbody(c, carry):
            off = wid * _RPW + c * _GCH
            pltpu.sync_copy(idx_hbm.at[pl.ds(off, _GCH)], idx_v)
            pltpu.async_copy(tbl_hbm.at[idx_v], rows_v, gsem).wait()
            pltpu.sync_copy(rows_v, h3_hbm.at[pl.ds(off, _GCH)])
            return carry
        lax.fori_loop(0, _GN, gbody, 0)

    return k(tbl, idx_all, tpad)


def _layer(H, C, W, cb, aW, ab, n, k):
    f32 = jnp.float32
    deg = jnp.sum(C, axis=1, keepdims=True) + 1.0
    dinv = 1.0 / jnp.sqrt(deg)
    rdeg = 1.0 / deg

    Hh = jnp.dot(H, W, preferred_element_type=f32)
    agg = dinv * jnp.dot(C, dinv * Hh, preferred_element_type=f32, precision=lax.Precision.HIGHEST)
    Hc = jnp.maximum(agg + Hh * rdeg + cb, 0.0)

    sh = jnp.dot(Hc, aW, preferred_element_type=f32)
    s2d = dinv * jnp.dot(C, dinv * sh, preferred_element_type=f32, precision=lax.Precision.HIGHEST) \
        + sh * rdeg + ab

    eyeN = (lax.broadcasted_iota(jnp.int32, (n, n), 0)
            == lax.broadcasted_iota(jnp.int32, (n, n), 1)).astype(f32)
    st = lax.dot_general(s2d, eyeN, (((0,), (0,)), ((), ())),
                         preferred_element_type=f32, precision=lax.Precision.HIGHEST)
    Sj = s2d * jnp.ones((n, n), f32)
    Si = st * jnp.ones((n, n), f32)
    ij = lax.broadcasted_iota(jnp.int32, (n, n), 0)
    ii = lax.broadcasted_iota(jnp.int32, (n, n), 1)
    T = (Sj > Si) | ((Sj == Si) & (ij < ii))
    rank = jnp.sum(T.astype(f32), axis=0, keepdims=True)

    P = (lax.broadcasted_iota(jnp.int32, (k, n), 0).astype(f32)
         == jnp.ones((k, 1), f32) * rank).astype(f32)

    sk = jnp.dot(P, s2d, preferred_element_type=f32, precision=lax.Precision.HIGHEST)
    Hn = jnp.dot(P, Hc, preferred_element_type=f32, precision=lax.Precision.HIGHEST) * jnp.tanh(sk)
    CP = lax.dot_general(C, P, (((1,), (1,)), ((), ())),
                         preferred_element_type=f32, precision=lax.Precision.HIGHEST)
    Cn = jnp.dot(P, CP, preferred_element_type=f32, precision=lax.Precision.HIGHEST)

    mx = jnp.max(Hn, axis=0, keepdims=True)
    mn = jnp.sum(Hn, axis=0, keepdims=True) / f32(k)
    return Hn, Cn, mx, mn


_GB = 4


def _gnn_body(ha, hb, hc, c2, c1W, c1b, a1W, a1b, c2W, c2b, a2W, a2b,
              c3W, c3b, a3W, a3b, out):
    Ws = ((c1W[...], c1b[...], a1W[...], a1b[...]),
          (c2W[...], c2b[...], a2W[...], a2b[...]),
          (c3W[...], c3b[...], a3W[...], a3b[...]))
    for g in range(_GB):
        H = ha[g] + hb[g] + hc[g]
        C = c2[g]
        n = _NPER
        acc = None
        for (cW, cb, aW, ab) in Ws:
            k = int(ceil(0.5 * n))
            H, C, mx, mn = _layer(H, C, cW, cb, aW, ab, n, k)
            r = jnp.concatenate([mx, mn], axis=1)
            acc = r if acc is None else acc + r
            n = k
        out[g] = acc


def _mlp_body(g, w1, b1, w2, b2, out):
    f32 = jnp.float32
    h = jnp.maximum(jnp.dot(g[...], w1[...], preferred_element_type=f32)
                    + b1[...], 0.0)
    out[...] = jnp.maximum(jnp.dot(h, w2[...], preferred_element_type=f32)
                           + b2[...], 0.0)


def _pred_body(gg, pW, pb, out):
    out[...] = (jnp.dot(gg[...], pW[0], preferred_element_type=jnp.float32)
                + pb[0])[None]


def kernel(x, node_depth, edge_index, edge_attr, batch, type_emb, attr_emb,
           depth_emb, conv1_W, conv1_b, attn1_W, attn1_b, conv2_W, conv2_b,
           attn2_W, attn2_b, conv3_W, conv3_b, attn3_W, attn3_b, lin1_W,
           lin1_b, lin2_W, lin2_b, pred_W, pred_b):
    f32 = jnp.float32
    i32 = jnp.int32

    src = edge_index[0].astype(i32)
    dst = edge_index[1].astype(i32)
    g = src // _NPER
    t = g * (_NPER * _NPER) + (dst % _NPER) * _NPER + (src % _NPER)
    tpad = jnp.full((_EPAD + 2 * _ECH,), _CSLOTS, i32).at[: _E].set(t)

    nt = type_emb.shape[0]
    na = attr_emb.shape[0]
    depth = jnp.clip(node_depth[:, 0], 0, _MAXDEPTH).astype(i32)
    i0 = jnp.zeros((_NPAD,), i32).at[: _N].set(x[:, 0].astype(i32))
    i1 = jnp.zeros((_NPAD,), i32).at[: _N].set(nt + x[:, 1].astype(i32))
    i2 = jnp.zeros((_NPAD,), i32).at[: _N].set(nt + na + depth)
    idx_all = jnp.concatenate([i0, i1, i2])
    tbl = jnp.concatenate([type_emb, attr_emb, depth_emb], axis=0)

    h3, c2 = _sc_gather_scatter(tbl, idx_all, tpad)

    ha = h3[0:_N].reshape(_B, _NPER, _D)
    hb = h3[_NPAD:_NPAD + _N].reshape(_B, _NPER, _D)
    hc = h3[2 * _NPAD:2 * _NPAD + _N].reshape(_B, _NPER, _D)
    c_lin = c2.reshape(_CPAD)[:_CSLOTS].reshape(_B, _NPER, _NPER)

    biases = [b.reshape(1, -1) for b in
              (conv1_b, attn1_b, conv2_b, attn2_b, conv3_b, attn3_b,
               lin1_b, lin2_b)]
    (c1b, a1b, c2b, a2b, c3b, a3b, l1b, l2b) = biases

    full = lambda arr: pl.BlockSpec(arr.shape, lambda gi: (0,) * arr.ndim)

    gsum = pl.pallas_call(
        _gnn_body,
        grid=(_B // _GB,),
        in_specs=[
            pl.BlockSpec((_GB, _NPER, _D), lambda gi: (gi, 0, 0)),
            pl.BlockSpec((_GB, _NPER, _D), lambda gi: (gi, 0, 0)),
            pl.BlockSpec((_GB, _NPER, _D), lambda gi: (gi, 0, 0)),
            pl.BlockSpec((_GB, _NPER, _NPER), lambda gi: (gi, 0, 0)),
            full(conv1_W), full(c1b), full(attn1_W), full(a1b),
            full(conv2_W), full(c2b), full(attn2_W), full(a2b),
            full(conv3_W), full(c3b), full(attn3_W), full(a3b),
        ],
        out_specs=pl.BlockSpec((_GB, 1, 2 * _D), lambda gi: (gi, 0, 0)),
        out_shape=jax.ShapeDtypeStruct((_B, 1, 2 * _D), f32),
    )(ha, hb, hc, c_lin, conv1_W, c1b, attn1_W, a1b,
      conv2_W, c2b, attn2_W, a2b, conv3_W, c3b, attn3_W, a3b)
    gsum = gsum.reshape(_B, 2 * _D)

    gg = pl.pallas_call(
        _mlp_body,
        out_shape=jax.ShapeDtypeStruct((_B, _D), f32),
    )(gsum, lin1_W, l1b, lin2_W, l2b)

    SEQ, _, VOCAB = pred_W.shape
    preds = pl.pallas_call(
        _pred_body,
        grid=(SEQ,),
        in_specs=[
            pl.BlockSpec((_B, _D), lambda s: (0, 0)),
            pl.BlockSpec((1, _D, VOCAB), lambda s: (s, 0, 0)),
            pl.BlockSpec((1, 1, VOCAB), lambda s: (s, 0, 0)),
        ],
        out_specs=pl.BlockSpec((1, _B, VOCAB), lambda s: (s, 0, 0)),
        out_shape=jax.ShapeDtypeStruct((SEQ, _B, VOCAB), f32),
    )(gg, pred_W, pred_b.reshape(SEQ, 1, VOCAB))

    return preds
